# baseline (speedup 1.0000x reference)
import numpy as np
import jax
import jax.numpy as jnp
from jax.experimental import pallas as pl
from jax.experimental.pallas import tpu as pltpu

_BS = 5
_PAD = _BS - 1
_B, _C, _H, _W = 8, 96, 224, 224
_MH, _MW = _H - _PAD, _W - _PAD
_NCH = _B * _C
_CH = _MH * _MW
_COUNT_M = _B * _C * _H * _W
_DH, _DW = _H + _PAD, _W + _PAD
_ROT_A = (13, 15, 26, 6)
_ROT_B = (17, 29, 16, 24)

_CPC1 = 12
_CPC2 = 32


def _threefry2x32(k1, k2, x0, x1):
    m = 0xFFFFFFFF
    k3 = k1 ^ k2 ^ 0x1BD11BDA
    sched = ((k2, (k3 + 1) & m), (k3, (k1 + 2) & m), (k1, (k2 + 3) & m),
             (k2, (k3 + 4) & m), (k3, (k1 + 5) & m))
    rots = (_ROT_A, _ROT_B, _ROT_A, _ROT_B, _ROT_A)
    x0 = x0 + np.uint32(k1)
    x1 = x1 + np.uint32(k2)
    for rset, (ka, kb) in zip(rots, sched):
        for r in rset:
            x0 = x0 + x1
            x1 = ((x1 << np.uint32(r)) | (x1 >> np.uint32(32 - r))) ^ x0
        x0 = x0 + np.uint32(ka)
        x1 = x1 + np.uint32(kb)
    return x0, x1


_o0, _o1 = _threefry2x32(0, 0, np.zeros(1, np.uint32), np.ones(1, np.uint32))
_KEY1, _KEY2 = int(_o0[0]), int(_o1[0])


def _mask_kernel(gamma_ref, packed_ref, cnt_ref):
    n = pl.program_id(0)
    g = gamma_ref[0, 0]
    thresh = jnp.ceil(g * jnp.float32(8388608.0)).astype(jnp.uint32)
    ts = jnp.minimum(thresh, jnp.uint32(8388607)) * jnp.uint32(512)
    shp = (_CPC1, _H, _DW)
    ch = jax.lax.broadcasted_iota(jnp.int32, shp, 0)
    a = jax.lax.broadcasted_iota(jnp.int32, shp, 1)
    b = jax.lax.broadcasted_iota(jnp.int32, shp, 2)
    c = b - _PAD
    lin = (n * _CPC1 + ch) * _CH + a * _MW + c
    o0, o1 = _threefry2x32(_KEY1, _KEY2,
                           jnp.zeros(shp, jnp.uint32), lin.astype(jnp.uint32))
    rbits = o0 ^ o1
    valid = ((a.astype(jnp.uint32) < jnp.uint32(_MH))
             & (c.astype(jnp.uint32) < jnp.uint32(_MW)))
    bern = valid & (rbits < ts)
    nw = _H // 32
    b4 = bern.reshape(_CPC1, nw, 32, _DW)
    sh = jax.lax.broadcasted_iota(jnp.uint32, (_CPC1, nw, 32, _DW), 2)
    p = jnp.where(b4, jnp.uint32(1) << sh, jnp.uint32(0))
    p = p[:, :, 0:16, :] | p[:, :, 16:32, :]
    p = p[:, :, 0:8, :] | p[:, :, 8:16, :]
    p = p[:, :, 0:4, :] | p[:, :, 4:8, :]
    p = p[:, :, 0:2, :] | p[:, :, 2:4, :]
    w = p[:, :, 0, :] | p[:, :, 1, :]
    c2 = w[:, :, 0:_W + 2] | w[:, :, 1:_W + 3]
    c4 = c2[:, :, 0:_W] | c2[:, :, 2:_W + 2]
    q = c4 | w[:, :, _PAD:_PAD + _W]
    z1 = jnp.zeros((_CPC1, 1, _W), jnp.uint32)
    qm1 = jnp.concatenate([z1, q[:, :nw - 1, :]], axis=1)
    h1 = q | ((q << np.uint32(1)) | (qm1 >> np.uint32(31)))
    hm1 = jnp.concatenate([z1, h1[:, :nw - 1, :]], axis=1)
    h2 = h1 | ((h1 << np.uint32(2)) | (hm1 >> np.uint32(30)))
    d = h2 | ((q << np.uint32(4)) | (qm1 >> np.uint32(28)))
    packed_ref[...] = d
    v = d - ((d >> np.uint32(1)) & np.uint32(0x55555555))
    v = (v & np.uint32(0x33333333)) + ((v >> np.uint32(2)) & np.uint32(0x33333333))
    v = (v + (v >> np.uint32(4))) & np.uint32(0x0F0F0F0F)
    v = (v + (v >> np.uint32(8)) + (v >> np.uint32(16)) + (v >> np.uint32(24))) & np.uint32(0xFF)
    cnt_step = jnp.sum(v.astype(jnp.int32))

    @pl.when(n == 0)
    def _():
        cnt_ref[0, 0] = jnp.int32(0)

    cnt_ref[0, 0] += cnt_step


def _apply_kernel(x_ref, packed_ref, cnt_ref, out_ref):
    dropped = cnt_ref[0, 0].astype(jnp.float32)
    scale = jnp.float32(_COUNT_M) / (jnp.float32(_COUNT_M) - dropped)
    w = packed_ref[...]
    sh = jax.lax.broadcasted_iota(jnp.uint32, (_CPC2, _H // 32, 32, _W), 2)
    bits = (w[:, :, None, :] >> sh) & np.uint32(1)
    drop = (bits != 0).reshape(_CPC2, _H, _W)
    out_ref[...] = jnp.where(drop, jnp.float32(0.0), x_ref[...] * scale)


def kernel(x, gamma):
    xr = x.reshape(_NCH, _H, _W)
    g2 = jnp.asarray(gamma, jnp.float32).reshape(1, 1)
    packed, cnt = pl.pallas_call(
        _mask_kernel,
        grid=(_NCH // _CPC1,),
        in_specs=[pl.BlockSpec((1, 1), lambda n: (0, 0), memory_space=pltpu.SMEM)],
        out_specs=[
            pl.BlockSpec((_CPC1, _H // 32, _W), lambda n: (n, 0, 0)),
            pl.BlockSpec((1, 1), lambda n: (0, 0), memory_space=pltpu.SMEM),
        ],
        out_shape=[
            jax.ShapeDtypeStruct((_NCH, _H // 32, _W), jnp.uint32),
            jax.ShapeDtypeStruct((1, 1), jnp.int32),
        ],
        compiler_params=pltpu.CompilerParams(
            dimension_semantics=("arbitrary",)),
    )(g2)
    out = pl.pallas_call(
        _apply_kernel,
        grid=(_NCH // _CPC2,),
        in_specs=[
            pl.BlockSpec((_CPC2, _H, _W), lambda n: (n, 0, 0)),
            pl.BlockSpec((_CPC2, _H // 32, _W), lambda n: (n, 0, 0)),
            pl.BlockSpec((1, 1), lambda n: (0, 0), memory_space=pltpu.SMEM),
        ],
        out_specs=pl.BlockSpec((_CPC2, _H, _W), lambda n: (n, 0, 0)),
        out_shape=jax.ShapeDtypeStruct((_NCH, _H, _W), x.dtype),
        compiler_params=pltpu.CompilerParams(
            dimension_semantics=("arbitrary",)),
    )(xr, packed, cnt)
    return out.reshape(_B, _C, _H, _W)


# speedup vs baseline: 1.0064x; 1.0064x over previous
import numpy as np
import jax
import jax.numpy as jnp
from jax.experimental import pallas as pl
from jax.experimental.pallas import tpu as pltpu

_BS = 5
_PAD = _BS - 1
_B, _C, _H, _W = 8, 96, 224, 224
_MH, _MW = _H - _PAD, _W - _PAD
_NCH = _B * _C
_CH = _MH * _MW
_COUNT_M = _B * _C * _H * _W
_DH, _DW = _H + _PAD, _W + _PAD
_ROT_A = (13, 15, 26, 6)
_ROT_B = (17, 29, 16, 24)

_CPC1 = 16
_CPC2 = 32


def _threefry2x32(k1, k2, x0, x1):
    m = 0xFFFFFFFF
    k3 = k1 ^ k2 ^ 0x1BD11BDA
    sched = ((k2, (k3 + 1) & m), (k3, (k1 + 2) & m), (k1, (k2 + 3) & m),
             (k2, (k3 + 4) & m), (k3, (k1 + 5) & m))
    rots = (_ROT_A, _ROT_B, _ROT_A, _ROT_B, _ROT_A)
    for rset, (ka, kb) in zip(rots, sched):
        for r in rset:
            x0 = x0 + x1
            x1 = ((x1 << np.uint32(r)) | (x1 >> np.uint32(32 - r))) ^ x0
        x0 = x0 + np.uint32(ka)
        x1 = x1 + np.uint32(kb)
    return x0, x1


_o0, _o1 = _threefry2x32(0, 0, np.zeros(1, np.uint32), np.ones(1, np.uint32))
_KEY1, _KEY2 = int(_o0[0]), int(_o1[0])


def _mask_tables():
    ch = np.arange(_CPC1, dtype=np.uint32).reshape(_CPC1, 1, 1)
    a = np.arange(_H, dtype=np.uint32).reshape(1, _H, 1)
    b = np.arange(_DW, dtype=np.uint32).reshape(1, 1, _DW)
    base = (ch * np.uint32(_CH) + a * np.uint32(_MW) + b
            - np.uint32(_PAD) + np.uint32(_KEY2)).astype(np.uint32)
    valid = (a < _MH) & (b >= _PAD) & (b < _PAD + _MW)
    k = np.arange(32, dtype=np.uint32).reshape(1, 1, 32, 1)
    oh = np.where(valid.reshape(1, _H // 32, 32, _DW),
                  np.uint32(1) << k, np.uint32(0)).astype(np.uint32)
    return base, oh


def _mask_kernel(gamma_ref, base_ref, oh_ref, packed_ref, cnt_ref):
    n = pl.program_id(0)
    g = gamma_ref[0, 0]
    thresh = jnp.ceil(g * jnp.float32(8388608.0)).astype(jnp.uint32)
    ts = jnp.minimum(thresh, jnp.uint32(8388607)) * jnp.uint32(512)
    shp = (_CPC1, _H, _DW)
    step = (n * (_CPC1 * _CH)).astype(jnp.uint32)
    x1 = base_ref[...] + step
    x0 = jnp.full(shp, jnp.uint32(_KEY1))
    o0, o1 = _threefry2x32(_KEY1, _KEY2, x0, x1)
    rbits = o0 ^ o1
    bern = rbits < ts
    nw = _H // 32
    b4 = bern.reshape(_CPC1, nw, 32, _DW)
    p = jnp.where(b4, oh_ref[...], jnp.uint32(0))
    p = p[:, :, 0:16, :] | p[:, :, 16:32, :]
    p = p[:, :, 0:8, :] | p[:, :, 8:16, :]
    p = p[:, :, 0:4, :] | p[:, :, 4:8, :]
    p = p[:, :, 0:2, :] | p[:, :, 2:4, :]
    w = p[:, :, 0, :] | p[:, :, 1, :]
    c2 = w[:, :, 0:_W + 2] | w[:, :, 1:_W + 3]
    c4 = c2[:, :, 0:_W] | c2[:, :, 2:_W + 2]
    q = c4 | w[:, :, _PAD:_PAD + _W]
    z1 = jnp.zeros((_CPC1, 1, _W), jnp.uint32)
    qm1 = jnp.concatenate([z1, q[:, :nw - 1, :]], axis=1)
    h1 = q | ((q << np.uint32(1)) | (qm1 >> np.uint32(31)))
    hm1 = jnp.concatenate([z1, h1[:, :nw - 1, :]], axis=1)
    h2 = h1 | ((h1 << np.uint32(2)) | (hm1 >> np.uint32(30)))
    d = h2 | ((q << np.uint32(4)) | (qm1 >> np.uint32(28)))
    packed_ref[...] = d
    v = d - ((d >> np.uint32(1)) & np.uint32(0x55555555))
    v = (v & np.uint32(0x33333333)) + ((v >> np.uint32(2)) & np.uint32(0x33333333))
    v = (v + (v >> np.uint32(4))) & np.uint32(0x0F0F0F0F)
    v = (v + (v >> np.uint32(8)) + (v >> np.uint32(16)) + (v >> np.uint32(24))) & np.uint32(0xFF)
    cnt_step = jnp.sum(v.astype(jnp.int32))

    @pl.when(n == 0)
    def _():
        cnt_ref[0, 0] = jnp.int32(0)

    cnt_ref[0, 0] += cnt_step


def _apply_kernel(x_ref, packed_ref, cnt_ref, out_ref):
    dropped = cnt_ref[0, 0].astype(jnp.float32)
    scale = jnp.float32(_COUNT_M) / (jnp.float32(_COUNT_M) - dropped)
    w = packed_ref[...]
    sh = jax.lax.broadcasted_iota(jnp.uint32, (_CPC2, _H // 32, 32, _W), 2)
    bits = (w[:, :, None, :] >> sh) & np.uint32(1)
    drop = (bits != 0).reshape(_CPC2, _H, _W)
    out_ref[...] = jnp.where(drop, jnp.float32(0.0), x_ref[...] * scale)


def kernel(x, gamma):
    xr = x.reshape(_NCH, _H, _W)
    g2 = jnp.asarray(gamma, jnp.float32).reshape(1, 1)
    base_np, oh_np = _mask_tables()
    base = jnp.asarray(base_np)
    oh = jnp.asarray(oh_np)
    nw = _H // 32
    packed, cnt = pl.pallas_call(
        _mask_kernel,
        grid=(_NCH // _CPC1,),
        in_specs=[
            pl.BlockSpec((1, 1), lambda n: (0, 0), memory_space=pltpu.SMEM),
            pl.BlockSpec((_CPC1, _H, _DW), lambda n: (0, 0, 0)),
            pl.BlockSpec((1, nw, 32, _DW), lambda n: (0, 0, 0, 0)),
        ],
        out_specs=[
            pl.BlockSpec((_CPC1, _H // 32, _W), lambda n: (n, 0, 0)),
            pl.BlockSpec((1, 1), lambda n: (0, 0), memory_space=pltpu.SMEM),
        ],
        out_shape=[
            jax.ShapeDtypeStruct((_NCH, _H // 32, _W), jnp.uint32),
            jax.ShapeDtypeStruct((1, 1), jnp.int32),
        ],
        compiler_params=pltpu.CompilerParams(
            dimension_semantics=("arbitrary",)),
    )(g2, base, oh)
    out = pl.pallas_call(
        _apply_kernel,
        grid=(_NCH // _CPC2,),
        in_specs=[
            pl.BlockSpec((_CPC2, _H, _W), lambda n: (n, 0, 0)),
            pl.BlockSpec((_CPC2, _H // 32, _W), lambda n: (n, 0, 0)),
            pl.BlockSpec((1, 1), lambda n: (0, 0), memory_space=pltpu.SMEM),
        ],
        out_specs=pl.BlockSpec((_CPC2, _H, _W), lambda n: (n, 0, 0)),
        out_shape=jax.ShapeDtypeStruct((_NCH, _H, _W), x.dtype),
        compiler_params=pltpu.CompilerParams(
            dimension_semantics=("arbitrary",)),
    )(xr, packed, cnt)
    return out.reshape(_B, _C, _H, _W)
